# baseline (speedup 1.0000x reference)
import jax
import jax.numpy as jnp
from jax import lax
from jax.experimental import pallas as pl
from jax.experimental.pallas import tpu as pltpu
from jax.experimental.pallas import tpu_sc as plsc

B = 4096
L = 200
D = 64
C = 2

NC = 2
NS = 16
NW = NC * NS
BPW = B // NW
HALF = L // 2
ROWS_I = 2 * BPW


def _sc_body(ids_hbm, table_hbm, out_hbm, ids_v, rows_v, out_v, sem):
    wid = lax.axis_index("s") * NC + lax.axis_index("c")
    base = wid * BPW
    pltpu.sync_copy(ids_hbm.at[pl.ds(base * 2, ROWS_I)], ids_v)

    def row_body(b, carry):
        def half_body(h, accs):
            pltpu.async_copy(table_hbm.at[ids_v.at[2 * b + h]], rows_v, sem).wait()

            def tok_body(t, accs):
                a0, a1, a2, a3 = accs
                a0 = a0 + rows_v[t, pl.ds(0, 16)]
                a1 = a1 + rows_v[t, pl.ds(16, 16)]
                a2 = a2 + rows_v[t, pl.ds(32, 16)]
                a3 = a3 + rows_v[t, pl.ds(48, 16)]
                return (a0, a1, a2, a3)

            return lax.fori_loop(0, HALF, tok_body, accs)

        z = jnp.zeros((16,), jnp.float32)
        a0, a1, a2, a3 = lax.fori_loop(0, 2, half_body, (z, z, z, z))
        out_v[b, pl.ds(0, 16)] = a0
        out_v[b, pl.ds(16, 16)] = a1
        out_v[b, pl.ds(32, 16)] = a2
        out_v[b, pl.ds(48, 16)] = a3
        return carry

    lax.fori_loop(0, BPW, row_body, 0)
    pltpu.sync_copy(out_v, out_hbm.at[pl.ds(base, BPW)])


_SC_CACHE = {}


def _sc_gather_sum_fn():
    if "k" not in _SC_CACHE:
        _SC_CACHE["k"] = pl.kernel(
            _sc_body,
            out_type=jax.ShapeDtypeStruct((B, D), jnp.float32),
            mesh=plsc.VectorSubcoreMesh(
                core_axis_name="c", subcore_axis_name="s",
                num_cores=NC, num_subcores=NS,
            ),
            scratch_types=[
                pltpu.VMEM((ROWS_I, HALF), jnp.int32),
                pltpu.VMEM((HALF, D), jnp.float32),
                pltpu.VMEM((BPW, D), jnp.float32),
                pltpu.SemaphoreType.DMA,
            ],
            compiler_params=pltpu.CompilerParams(use_tc_tiling_on_sc=False),
        )
    return _SC_CACHE["k"]

BB = 512


def _tc_body(ids_ref, sums_ref, w_ref, b_ref, out_ref):
    ids = ids_ref[...]
    cnt = jnp.sum((ids != 0).astype(jnp.float32), axis=1, keepdims=True)
    sent = sums_ref[...] / (cnt + 1e-8)
    out_ref[...] = (
        lax.dot_general(sent, w_ref[...], (((1,), (1,)), ((), ())),
                        preferred_element_type=jnp.float32)
        + b_ref[...]
    )


_tc_head = pl.pallas_call(
    _tc_body,
    grid=(B // BB,),
    in_specs=[
        pl.BlockSpec((BB, L), lambda i: (i, 0)),
        pl.BlockSpec((BB, D), lambda i: (i, 0)),
        pl.BlockSpec((C, D), lambda i: (0, 0)),
        pl.BlockSpec((1, C), lambda i: (0, 0)),
    ],
    out_specs=pl.BlockSpec((BB, C), lambda i: (i, 0)),
    out_shape=jax.ShapeDtypeStruct((B, C), jnp.float32),
)


def kernel(input_ids, table, W, b):
    ids = input_ids.astype(jnp.int32)
    ids2 = ids.reshape(B * 2, HALF)
    sums = _sc_gather_sum_fn()(ids2, table)
    return _tc_head(ids, sums, W, b.reshape(1, C))
